# Initial kernel scaffold; baseline (speedup 1.0000x reference)
import jax
import jax.numpy as jnp
from jax.experimental import pallas as pl


def kernel(entity, edge_index, edge_attr, emb_table, W1, att1, b1, W2, att2, b2):
    raise NotImplementedError("write your pallas kernel here")


# speedup vs baseline: 3.5038x; 3.5038x over previous
import functools

import jax
import jax.numpy as jnp
from jax import lax
from jax.experimental import pallas as pl
from jax.experimental.pallas import tpu as pltpu
from jax.experimental.pallas import tpu_sc as plsc

N = 10000
E = 320000
D = 128
H1 = 8
NREL = 401
UW = 144

NC = 2
NS = 16
LANES = 16


_EMB_B = 10240
_EMB_PER_W = _EMB_B // (NC * NS)


def _emb_body(table, idx, out, idx_v, rows_v, sem):
    wid = lax.axis_index("s") * NC + lax.axis_index("c")
    base = wid * _EMB_PER_W
    pltpu.sync_copy(idx.at[pl.ds(base, _EMB_PER_W)], idx_v)
    pltpu.async_copy(table.at[idx_v], rows_v, sem).wait()
    pltpu.sync_copy(rows_v, out.at[pl.ds(base, _EMB_PER_W)])


def _emb_gather(table, idx_padded):
    mesh = plsc.VectorSubcoreMesh(core_axis_name="c", subcore_axis_name="s")
    return pl.kernel(
        _emb_body,
        out_type=jax.ShapeDtypeStruct((_EMB_B, D), jnp.float32),
        mesh=mesh,
        scratch_types=[
            pltpu.VMEM((_EMB_PER_W,), jnp.int32),
            pltpu.VMEM((_EMB_PER_W, D), jnp.float32),
            pltpu.SemaphoreType.DMA,
        ],
        compiler_params=pltpu.CompilerParams(
            needs_layout_passes=False, use_tc_tiling_on_sc=False),
    )(table, idx_padded)


CHUNK = 16
EPT = E // (NC * NS)
NCHUNK = EPT // CHUNK
NP = 10240
ROWS_PER_S = NP // NS


def _edge_body(xtab, atab, srcr, dstr, ear, zerosr, out,
               s_v, d_v, e_v, xi_v, xj_v, at_v, msg_v, pt_v, ee_v, U_sh,
               sem1, sem2, sem3):
    c = lax.axis_index("c")
    s = lax.axis_index("s")
    tid = c * NS + s
    iota = jnp.arange(LANES, dtype=jnp.int32)
    unit = (iota == 0).astype(jnp.float32)

    r0 = s * ROWS_PER_S
    pltpu.sync_copy(zerosr.at[pl.ds(r0, ROWS_PER_S)],
                    U_sh.at[pl.ds(r0, ROWS_PER_S)])
    plsc.subcore_barrier()

    base_edge = tid * EPT

    def body(i, _):
        off = base_edge + i * CHUNK
        pltpu.sync_copy(srcr.at[pl.ds(off, CHUNK)], s_v)
        pltpu.sync_copy(dstr.at[pl.ds(off, CHUNK)], d_v)
        pltpu.sync_copy(ear.at[pl.ds(off, CHUNK)], e_v)
        cp1 = pltpu.async_copy(xtab.at[d_v], xi_v, sem1)
        cp2 = pltpu.async_copy(xtab.at[s_v], xj_v, sem2)
        cp3 = pltpu.async_copy(atab.at[e_v], at_v, sem3)
        cp1.wait()
        cp2.wait()
        cp3.wait()
        for e in range(CHUNK):
            acc = xi_v[e, pl.ds(0, 16)] * at_v[e, pl.ds(0, 16)] \
                * xj_v[e, pl.ds(0, 16)]
            for k in range(1, 8):
                acc = acc + xi_v[e, pl.ds(k * 16, 16)] \
                    * at_v[e, pl.ds(k * 16, 16)] * xj_v[e, pl.ds(k * 16, 16)]
            plsc.store_scatter(pt_v, [iota, jnp.full((LANES,), e, jnp.int32)],
                               acc)
        alpha = pt_v[0, :]
        for k in range(1, 16):
            alpha = alpha + pt_v[k, :]
        alpha = jnp.where(alpha > 0, alpha, 0.2 * alpha)
        ee_v[:] = jnp.exp(alpha)
        for e in range(CHUNK):
            eb = plsc.load_gather(ee_v, [jnp.full((LANES,), e, jnp.int32)])
            for k in range(8):
                msg_v[e, pl.ds(k * 16, 16)] = eb * xj_v[e, pl.ds(k * 16, 16)]
            msg_v[e, pl.ds(128, 16)] = eb * unit
        pltpu.sync_copy(msg_v, U_sh.at[d_v], add=True)
        return ()

    lax.fori_loop(0, NCHUNK, body, ())
    plsc.subcore_barrier()
    pltpu.sync_copy(U_sh.at[pl.ds(r0, ROWS_PER_S)],
                    out.at[pl.ds(c * NP + r0, ROWS_PER_S)])


def _edge_pass(xtab, atab, src, dst, ea, zeros_nuw):
    mesh = plsc.VectorSubcoreMesh(core_axis_name="c", subcore_axis_name="s")
    return pl.kernel(
        _edge_body,
        out_type=jax.ShapeDtypeStruct((2 * NP, UW), jnp.float32),
        mesh=mesh,
        scratch_types=[
            pltpu.VMEM((CHUNK,), jnp.int32),
            pltpu.VMEM((CHUNK,), jnp.int32),
            pltpu.VMEM((CHUNK,), jnp.int32),
            pltpu.VMEM((CHUNK, D), jnp.float32),
            pltpu.VMEM((CHUNK, D), jnp.float32),
            pltpu.VMEM((CHUNK, D), jnp.float32),
            pltpu.VMEM((CHUNK, UW), jnp.float32),
            pltpu.VMEM((LANES, LANES), jnp.float32),
            pltpu.VMEM((LANES,), jnp.float32),
            pltpu.VMEM_SHARED((NP, UW), jnp.float32),
            pltpu.SemaphoreType.DMA,
            pltpu.SemaphoreType.DMA,
            pltpu.SemaphoreType.DMA,
        ],
        compiler_params=pltpu.CompilerParams(
            needs_layout_passes=False, use_tc_tiling_on_sc=False),
    )(xtab, atab, src, dst, ea, zeros_nuw)


_BLK1 = 2000


def _mm1_body(x0_ref, w_ref, aself_ref, x1_ref, ui_ref):
    x0 = x0_ref[...]
    w = w_ref[0]
    y = jnp.dot(x0, w, preferred_element_type=jnp.float32)
    a = aself_ref[0]
    alpha = jnp.sum(y * y * a, axis=-1)
    alpha = jnp.where(alpha > 0, alpha, 0.2 * alpha)
    ee = jnp.exp(alpha)[:, None]
    x1_ref[0] = y
    ui_ref[0, :, :D] = ee * y
    lane16 = (lax.broadcasted_iota(jnp.int32, (1, 16), 1) == 0)
    ui_ref[0, :, D:UW] = ee * lane16.astype(jnp.float32)


def _mm1(x0, W1r, a1self):
    nb = N // _BLK1
    return pl.pallas_call(
        _mm1_body,
        grid=(H1, nb),
        in_specs=[
            pl.BlockSpec((_BLK1, D), lambda h, i: (i, 0)),
            pl.BlockSpec((1, D, D), lambda h, i: (h, 0, 0)),
            pl.BlockSpec((1, 1, D), lambda h, i: (h, 0, 0)),
        ],
        out_specs=[
            pl.BlockSpec((1, _BLK1, D), lambda h, i: (h, i, 0)),
            pl.BlockSpec((1, _BLK1, UW), lambda h, i: (h, i, 0)),
        ],
        out_shape=[
            jax.ShapeDtypeStruct((H1, N, D), jnp.float32),
            jax.ShapeDtypeStruct((H1, N, UW), jnp.float32),
        ],
    )(x0, W1r, a1self)


_BLK2 = 400


def _mid_body(ua_ref, ub_ref, ui_ref, b1_ref, w2_ref, a2_ref,
              x2_ref, ui2_ref):
    x2 = jnp.zeros((_BLK2, D), dtype=jnp.float32)
    for h in range(H1):
        S = ua_ref[h] + ub_ref[h] + ui_ref[h]
        num = S[:, :D]
        den = S[:, D:D + 1]
        o = num / (den + 1e-16) + b1_ref[h][None, :]
        hh = jnp.where(o > 0, o, 0.01 * o)
        x2 = x2 + jnp.dot(hh, w2_ref[h], preferred_element_type=jnp.float32)
    a2 = a2_ref[...]
    alpha = jnp.sum(x2 * x2 * a2, axis=-1)
    alpha = jnp.where(alpha > 0, alpha, 0.2 * alpha)
    ee = jnp.exp(alpha)[:, None]
    x2_ref[...] = x2
    ui2_ref[:, :D] = ee * x2
    lane16 = (lax.broadcasted_iota(jnp.int32, (1, 16), 1) == 0)
    ui2_ref[:, D:UW] = ee * lane16.astype(jnp.float32)


def _mid(Ua, Ub, uinit1, b1r, W2r, a2self):
    nb = N // _BLK2
    return pl.pallas_call(
        _mid_body,
        grid=(nb,),
        in_specs=[
            pl.BlockSpec((H1, _BLK2, UW), lambda i: (0, i, 0)),
            pl.BlockSpec((H1, _BLK2, UW), lambda i: (0, i, 0)),
            pl.BlockSpec((H1, _BLK2, UW), lambda i: (0, i, 0)),
            pl.BlockSpec((H1, D), lambda i: (0, 0)),
            pl.BlockSpec((H1, D, D), lambda i: (0, 0, 0)),
            pl.BlockSpec((1, D), lambda i: (0, 0)),
        ],
        out_specs=[
            pl.BlockSpec((_BLK2, D), lambda i: (i, 0)),
            pl.BlockSpec((_BLK2, UW), lambda i: (i, 0)),
        ],
        out_shape=[
            jax.ShapeDtypeStruct((N, D), jnp.float32),
            jax.ShapeDtypeStruct((N, UW), jnp.float32),
        ],
    )(Ua, Ub, uinit1, b1r, W2r, a2self)


_BLK3 = 2000


def _fin_body(ua_ref, ub_ref, ui_ref, b2_ref, out_ref):
    S = ua_ref[...] + ub_ref[...] + ui_ref[...]
    num = S[:, :D]
    den = S[:, D:D + 1]
    out_ref[...] = num / (den + 1e-16) + b2_ref[...]


def _fin(Ua, Ub, uinit2, b2):
    nb = N // _BLK3
    return pl.pallas_call(
        _fin_body,
        grid=(nb,),
        in_specs=[
            pl.BlockSpec((_BLK3, UW), lambda i: (i, 0)),
            pl.BlockSpec((_BLK3, UW), lambda i: (i, 0)),
            pl.BlockSpec((_BLK3, UW), lambda i: (i, 0)),
            pl.BlockSpec((1, D), lambda i: (0, 0)),
        ],
        out_specs=pl.BlockSpec((_BLK3, D), lambda i: (i, 0)),
        out_shape=jax.ShapeDtypeStruct((N, D), jnp.float32),
    )(Ua, Ub, uinit2, b2)


def kernel(entity, edge_index, edge_attr, emb_table, W1, att1, b1, W2, att2,
           b2):
    src = edge_index[0]
    dst = edge_index[1]
    ea = edge_attr
    zeros_nuw = jnp.zeros((NP, UW), dtype=jnp.float32)

    idx_pad = jnp.pad(entity, (0, _EMB_B - N))
    x0 = _emb_gather(emb_table, idx_pad)[:N]

    W1r = W1.reshape(D, H1, D).transpose(1, 0, 2)
    a1self = att1[NREL - 1].reshape(H1, 1, D)
    x1h, uinit1 = _mm1(x0, W1r, a1self)

    att1h = att1.transpose(1, 0, 2)
    U1 = [_edge_pass(x1h[h], att1h[h], src, dst, ea, zeros_nuw)
          for h in range(H1)]
    Ua = jnp.stack([u[:N] for u in U1])
    Ub = jnp.stack([u[NP:NP + N] for u in U1])

    b1r = b1.reshape(H1, D)
    W2r = W2.reshape(H1, D, D)
    a2self = att2[NREL - 1]
    x2, uinit2 = _mid(Ua, Ub, uinit1, b1r, W2r, a2self)

    U2 = _edge_pass(x2, att2[:, 0, :], src, dst, ea, zeros_nuw)
    out = _fin(U2[:N], U2[NP:NP + N], uinit2, b2.reshape(1, D))
    return out


# speedup vs baseline: 5.5575x; 1.5862x over previous
import functools

import jax
import jax.numpy as jnp
from jax import lax
from jax.experimental import pallas as pl
from jax.experimental.pallas import tpu as pltpu
from jax.experimental.pallas import tpu_sc as plsc

N = 10000
E = 320000
D = 128
H1 = 8
NREL = 401
UW = 144

NC = 2
NS = 16
LANES = 16


_EMB_B = 10240
_EMB_PER_W = _EMB_B // (NC * NS)


def _emb_body(table, idx, out, idx_v, rows_v, sem):
    wid = lax.axis_index("s") * NC + lax.axis_index("c")
    base = wid * _EMB_PER_W
    pltpu.sync_copy(idx.at[pl.ds(base, _EMB_PER_W)], idx_v)
    pltpu.async_copy(table.at[idx_v], rows_v, sem).wait()
    pltpu.sync_copy(rows_v, out.at[pl.ds(base, _EMB_PER_W)])


def _emb_gather(table, idx_padded):
    mesh = plsc.VectorSubcoreMesh(core_axis_name="c", subcore_axis_name="s")
    return pl.kernel(
        _emb_body,
        out_type=jax.ShapeDtypeStruct((_EMB_B, D), jnp.float32),
        mesh=mesh,
        scratch_types=[
            pltpu.VMEM((_EMB_PER_W,), jnp.int32),
            pltpu.VMEM((_EMB_PER_W, D), jnp.float32),
            pltpu.SemaphoreType.DMA,
        ],
        compiler_params=pltpu.CompilerParams(
            needs_layout_passes=False, use_tc_tiling_on_sc=False),
    )(table, idx_padded)


CHUNK = 32
EPT = E // (NC * NS)
EPT_PAD = 10016
NCHUNK = EPT_PAD // CHUNK
NGRP = CHUNK // LANES
NP = 10240
ROWS_PER_S = NP // NS


def _edge_body(xtab, atab, idx3, zerosr, out,
               ixA, ixB, xi0, xj0, at0, xi1, xj1, at1,
               msg_v, pt_v, ee_v, U_sh,
               s1, s2, s3, s4, s5, s6, si):
    c = lax.axis_index("c")
    s = lax.axis_index("s")
    tid = c * NS + s
    iota = jnp.arange(LANES, dtype=jnp.int32)
    unit = (iota == 0).astype(jnp.float32)

    r0 = s * ROWS_PER_S
    pltpu.sync_copy(zerosr.at[pl.ds(r0, ROWS_PER_S)],
                    U_sh.at[pl.ds(r0, ROWS_PER_S)])
    plsc.subcore_barrier()

    rbase = tid * NCHUNK

    def issue(ix, xi_v, xj_v, at_v, sa, sb, sc_):
        cp1 = pltpu.async_copy(xtab.at[ix.at[1]], xi_v, sa)
        cp2 = pltpu.async_copy(xtab.at[ix.at[0]], xj_v, sb)
        cp3 = pltpu.async_copy(atab.at[ix.at[2]], at_v, sc_)
        return cp1, cp2, cp3

    def compute_scatter(ix, xi_v, xj_v, at_v):
        def grp(g, _):
            e0 = g * LANES
            for e in range(LANES):
                acc = xi_v[e0 + e, pl.ds(0, 16)] * at_v[e0 + e, pl.ds(0, 16)] \
                    * xj_v[e0 + e, pl.ds(0, 16)]
                for k in range(1, 8):
                    acc = acc + xi_v[e0 + e, pl.ds(k * 16, 16)] \
                        * at_v[e0 + e, pl.ds(k * 16, 16)] \
                        * xj_v[e0 + e, pl.ds(k * 16, 16)]
                plsc.store_scatter(
                    pt_v, [iota, jnp.full((LANES,), e, jnp.int32)], acc)
            alpha = pt_v[0, :]
            for k in range(1, 16):
                alpha = alpha + pt_v[k, :]
            alpha = jnp.where(alpha > 0, alpha, 0.2 * alpha)
            ee_v[:] = jnp.exp(alpha)
            for e in range(LANES):
                eb = plsc.load_gather(ee_v, [jnp.full((LANES,), e, jnp.int32)])
                for k in range(8):
                    msg_v[e0 + e, pl.ds(k * 16, 16)] = \
                        eb * xj_v[e0 + e, pl.ds(k * 16, 16)]
                msg_v[e0 + e, pl.ds(128, 16)] = eb * unit
            return ()

        lax.fori_loop(0, NGRP, grp, ())
        pltpu.sync_copy(msg_v, U_sh.at[ix.at[1]], add=True)

    pltpu.sync_copy(idx3.at[rbase], ixA)

    def body2(k, _):
        j0 = 2 * k
        cpi = pltpu.async_copy(idx3.at[rbase + j0 + 1], ixB, si)
        dA = issue(ixA, xi0, xj0, at0, s1, s2, s3)
        cpi.wait()
        dB = issue(ixB, xi1, xj1, at1, s4, s5, s6)
        dA[0].wait()
        dA[1].wait()
        dA[2].wait()
        compute_scatter(ixA, xi0, xj0, at0)
        cpi2 = pltpu.async_copy(idx3.at[rbase + j0 + 2], ixA, si)
        dB[0].wait()
        dB[1].wait()
        dB[2].wait()
        compute_scatter(ixB, xi1, xj1, at1)
        cpi2.wait()
        return ()

    lax.fori_loop(0, NCHUNK // 2, body2, ())
    dA = issue(ixA, xi0, xj0, at0, s1, s2, s3)
    dA[0].wait()
    dA[1].wait()
    dA[2].wait()
    compute_scatter(ixA, xi0, xj0, at0)

    plsc.subcore_barrier()
    pltpu.sync_copy(U_sh.at[pl.ds(r0, ROWS_PER_S)],
                    out.at[pl.ds(c * NP + r0, ROWS_PER_S)])


def _edge_pass(xtab, atab, idx3, zeros_nuw):
    mesh = plsc.VectorSubcoreMesh(core_axis_name="c", subcore_axis_name="s")
    return pl.kernel(
        _edge_body,
        out_type=jax.ShapeDtypeStruct((2 * NP, UW), jnp.float32),
        mesh=mesh,
        scratch_types=[
            pltpu.VMEM((3, CHUNK), jnp.int32),
            pltpu.VMEM((3, CHUNK), jnp.int32),
            pltpu.VMEM((CHUNK, D), jnp.float32),
            pltpu.VMEM((CHUNK, D), jnp.float32),
            pltpu.VMEM((CHUNK, D), jnp.float32),
            pltpu.VMEM((CHUNK, D), jnp.float32),
            pltpu.VMEM((CHUNK, D), jnp.float32),
            pltpu.VMEM((CHUNK, D), jnp.float32),
            pltpu.VMEM((CHUNK, UW), jnp.float32),
            pltpu.VMEM((LANES, LANES), jnp.float32),
            pltpu.VMEM((LANES,), jnp.float32),
            pltpu.VMEM_SHARED((NP, UW), jnp.float32),
            pltpu.SemaphoreType.DMA,
            pltpu.SemaphoreType.DMA,
            pltpu.SemaphoreType.DMA,
            pltpu.SemaphoreType.DMA,
            pltpu.SemaphoreType.DMA,
            pltpu.SemaphoreType.DMA,
            pltpu.SemaphoreType.DMA,
        ],
        compiler_params=pltpu.CompilerParams(
            needs_layout_passes=False, use_tc_tiling_on_sc=False),
    )(xtab, atab, idx3, zeros_nuw)


_BLK1 = 2000


def _mm1_body(x0_ref, w_ref, aself_ref, x1_ref, ui_ref):
    x0 = x0_ref[...]
    w = w_ref[0]
    y = jnp.dot(x0, w, preferred_element_type=jnp.float32)
    a = aself_ref[0]
    alpha = jnp.sum(y * y * a, axis=-1)
    alpha = jnp.where(alpha > 0, alpha, 0.2 * alpha)
    ee = jnp.exp(alpha)[:, None]
    x1_ref[0] = y
    ui_ref[0, :, :D] = ee * y
    lane16 = (lax.broadcasted_iota(jnp.int32, (1, 16), 1) == 0)
    ui_ref[0, :, D:UW] = ee * lane16.astype(jnp.float32)


def _mm1(x0, W1r, a1self):
    nb = N // _BLK1
    return pl.pallas_call(
        _mm1_body,
        grid=(H1, nb),
        in_specs=[
            pl.BlockSpec((_BLK1, D), lambda h, i: (i, 0)),
            pl.BlockSpec((1, D, D), lambda h, i: (h, 0, 0)),
            pl.BlockSpec((1, 1, D), lambda h, i: (h, 0, 0)),
        ],
        out_specs=[
            pl.BlockSpec((1, _BLK1, D), lambda h, i: (h, i, 0)),
            pl.BlockSpec((1, _BLK1, UW), lambda h, i: (h, i, 0)),
        ],
        out_shape=[
            jax.ShapeDtypeStruct((H1, N, D), jnp.float32),
            jax.ShapeDtypeStruct((H1, N, UW), jnp.float32),
        ],
    )(x0, W1r, a1self)


_BLK2 = 400


def _mid_body(ua_ref, ub_ref, ui_ref, b1_ref, w2_ref, a2_ref,
              x2_ref, ui2_ref):
    x2 = jnp.zeros((_BLK2, D), dtype=jnp.float32)
    for h in range(H1):
        S = ua_ref[h] + ub_ref[h] + ui_ref[h]
        num = S[:, :D]
        den = S[:, D:D + 1]
        o = num / (den + 1e-16) + b1_ref[h][None, :]
        hh = jnp.where(o > 0, o, 0.01 * o)
        x2 = x2 + jnp.dot(hh, w2_ref[h], preferred_element_type=jnp.float32)
    a2 = a2_ref[...]
    alpha = jnp.sum(x2 * x2 * a2, axis=-1)
    alpha = jnp.where(alpha > 0, alpha, 0.2 * alpha)
    ee = jnp.exp(alpha)[:, None]
    x2_ref[...] = x2
    ui2_ref[:, :D] = ee * x2
    lane16 = (lax.broadcasted_iota(jnp.int32, (1, 16), 1) == 0)
    ui2_ref[:, D:UW] = ee * lane16.astype(jnp.float32)


def _mid(Ua, Ub, uinit1, b1r, W2r, a2self):
    nb = N // _BLK2
    return pl.pallas_call(
        _mid_body,
        grid=(nb,),
        in_specs=[
            pl.BlockSpec((H1, _BLK2, UW), lambda i: (0, i, 0)),
            pl.BlockSpec((H1, _BLK2, UW), lambda i: (0, i, 0)),
            pl.BlockSpec((H1, _BLK2, UW), lambda i: (0, i, 0)),
            pl.BlockSpec((H1, D), lambda i: (0, 0)),
            pl.BlockSpec((H1, D, D), lambda i: (0, 0, 0)),
            pl.BlockSpec((1, D), lambda i: (0, 0)),
        ],
        out_specs=[
            pl.BlockSpec((_BLK2, D), lambda i: (i, 0)),
            pl.BlockSpec((_BLK2, UW), lambda i: (i, 0)),
        ],
        out_shape=[
            jax.ShapeDtypeStruct((N, D), jnp.float32),
            jax.ShapeDtypeStruct((N, UW), jnp.float32),
        ],
    )(Ua, Ub, uinit1, b1r, W2r, a2self)


_BLK3 = 2000


def _fin_body(ua_ref, ub_ref, ui_ref, b2_ref, out_ref):
    S = ua_ref[...] + ub_ref[...] + ui_ref[...]
    num = S[:, :D]
    den = S[:, D:D + 1]
    out_ref[...] = num / (den + 1e-16) + b2_ref[...]


def _fin(Ua, Ub, uinit2, b2):
    nb = N // _BLK3
    return pl.pallas_call(
        _fin_body,
        grid=(nb,),
        in_specs=[
            pl.BlockSpec((_BLK3, UW), lambda i: (i, 0)),
            pl.BlockSpec((_BLK3, UW), lambda i: (i, 0)),
            pl.BlockSpec((_BLK3, UW), lambda i: (i, 0)),
            pl.BlockSpec((1, D), lambda i: (0, 0)),
        ],
        out_specs=pl.BlockSpec((_BLK3, D), lambda i: (i, 0)),
        out_shape=jax.ShapeDtypeStruct((N, D), jnp.float32),
    )(Ua, Ub, uinit2, b2)


def kernel(entity, edge_index, edge_attr, emb_table, W1, att1, b1, W2, att2,
           b2):
    npad = EPT_PAD - EPT
    src = jnp.pad(edge_index[0].reshape(NC * NS, EPT), ((0, 0), (0, npad)),
                  constant_values=0).reshape(NC * NS, NCHUNK, CHUNK)
    dst = jnp.pad(edge_index[1].reshape(NC * NS, EPT), ((0, 0), (0, npad)),
                  constant_values=N).reshape(NC * NS, NCHUNK, CHUNK)
    ea = jnp.pad(edge_attr.reshape(NC * NS, EPT), ((0, 0), (0, npad)),
                 constant_values=0).reshape(NC * NS, NCHUNK, CHUNK)
    idx3 = jnp.stack([src, dst, ea], axis=2).reshape(
        NC * NS * NCHUNK, 3, CHUNK)
    zeros_nuw = jnp.zeros((NP, UW), dtype=jnp.float32)

    idx_pad = jnp.pad(entity, (0, _EMB_B - N))
    x0 = _emb_gather(emb_table, idx_pad)[:N]

    W1r = W1.reshape(D, H1, D).transpose(1, 0, 2)
    a1self = att1[NREL - 1].reshape(H1, 1, D)
    x1h, uinit1 = _mm1(x0, W1r, a1self)

    att1h = att1.transpose(1, 0, 2)
    U1 = [_edge_pass(x1h[h], att1h[h], idx3, zeros_nuw)
          for h in range(H1)]
    Ua = jnp.stack([u[:N] for u in U1])
    Ub = jnp.stack([u[NP:NP + N] for u in U1])

    b1r = b1.reshape(H1, D)
    W2r = W2.reshape(H1, D, D)
    a2self = att2[NREL - 1]
    x2, uinit2 = _mid(Ua, Ub, uinit1, b1r, W2r, a2self)

    U2 = _edge_pass(x2, att2[:, 0, :], idx3, zeros_nuw)
    out = _fin(U2[:N], U2[NP:NP + N], uinit2, b2.reshape(1, D))
    return out


# speedup vs baseline: 5.8085x; 1.0452x over previous
import functools

import jax
import jax.numpy as jnp
from jax import lax
from jax.experimental import pallas as pl
from jax.experimental.pallas import tpu as pltpu
from jax.experimental.pallas import tpu_sc as plsc

N = 10000
E = 320000
D = 128
H1 = 8
NREL = 401
UW = 144

NC = 2
NS = 16
LANES = 16


_EMB_B = 10240
_EMB_PER_W = _EMB_B // (NC * NS)


def _emb_body(table, idx, out, idx_v, rows_v, sem):
    wid = lax.axis_index("s") * NC + lax.axis_index("c")
    base = wid * _EMB_PER_W
    pltpu.sync_copy(idx.at[pl.ds(base, _EMB_PER_W)], idx_v)
    pltpu.async_copy(table.at[idx_v], rows_v, sem).wait()
    pltpu.sync_copy(rows_v, out.at[pl.ds(base, _EMB_PER_W)])


def _emb_gather(table, idx_padded):
    mesh = plsc.VectorSubcoreMesh(core_axis_name="c", subcore_axis_name="s")
    return pl.kernel(
        _emb_body,
        out_type=jax.ShapeDtypeStruct((_EMB_B, D), jnp.float32),
        mesh=mesh,
        scratch_types=[
            pltpu.VMEM((_EMB_PER_W,), jnp.int32),
            pltpu.VMEM((_EMB_PER_W, D), jnp.float32),
            pltpu.SemaphoreType.DMA,
        ],
        compiler_params=pltpu.CompilerParams(
            needs_layout_passes=False, use_tc_tiling_on_sc=False),
    )(table, idx_padded)


CHUNK = 32
EPT = E // (NC * NS)
EPT_PAD = 10016
NCHUNK = EPT_PAD // CHUNK
NGRP = CHUNK // LANES
NP = 10240
ROWS_PER_S = NP // NS


def _edge_body(xtab, atab, idx3, zerosr, out,
               ixA, ixB, xi0, xj0, at0, xi1, xj1, at1,
               msg0, msg1, pt_v, ee_v, U_sh, atab_sh,
               s1, s2, s3, s4, s5, s6, si, sm):
    c = lax.axis_index("c")
    s = lax.axis_index("s")
    tid = c * NS + s
    iota = jnp.arange(LANES, dtype=jnp.int32)
    unit = (iota == 0).astype(jnp.float32)

    r0 = s * ROWS_PER_S
    pltpu.sync_copy(zerosr.at[pl.ds(r0, ROWS_PER_S)],
                    U_sh.at[pl.ds(r0, ROWS_PER_S)])

    @pl.when(s == 0)
    def _():
        pltpu.sync_copy(atab, atab_sh)

    plsc.subcore_barrier()

    rbase = tid * NCHUNK

    def issue(ix, xi_v, xj_v, at_v, sa, sb, sc_):
        cp1 = pltpu.async_copy(xtab.at[ix.at[1]], xi_v, sa)
        cp2 = pltpu.async_copy(xtab.at[ix.at[0]], xj_v, sb)
        cp3 = pltpu.async_copy(atab_sh.at[ix.at[2]], at_v, sc_)
        return cp1, cp2, cp3

    def compute_scatter(ix, xi_v, xj_v, at_v, msg_v):
        def grp(g, _):
            e0 = g * LANES
            for e in range(LANES):
                acc = xi_v[e0 + e, pl.ds(0, 16)] * at_v[e0 + e, pl.ds(0, 16)] \
                    * xj_v[e0 + e, pl.ds(0, 16)]
                for k in range(1, 8):
                    acc = acc + xi_v[e0 + e, pl.ds(k * 16, 16)] \
                        * at_v[e0 + e, pl.ds(k * 16, 16)] \
                        * xj_v[e0 + e, pl.ds(k * 16, 16)]
                plsc.store_scatter(
                    pt_v, [iota, jnp.full((LANES,), e, jnp.int32)], acc)
            alpha = pt_v[0, :]
            for k in range(1, 16):
                alpha = alpha + pt_v[k, :]
            alpha = jnp.where(alpha > 0, alpha, 0.2 * alpha)
            ee_v[:] = jnp.exp(alpha)
            for e in range(LANES):
                eb = plsc.load_gather(ee_v, [jnp.full((LANES,), e, jnp.int32)])
                for k in range(8):
                    msg_v[e0 + e, pl.ds(k * 16, 16)] = \
                        eb * xj_v[e0 + e, pl.ds(k * 16, 16)]
                msg_v[e0 + e, pl.ds(128, 16)] = eb * unit
            return ()

        lax.fori_loop(0, NGRP, grp, ())
        return pltpu.async_copy(msg_v, U_sh.at[ix.at[1]], sm, add=True)

    pltpu.sync_copy(idx3.at[rbase], ixA)

    def body2(k, _):
        j0 = 2 * k
        cpi = pltpu.async_copy(idx3.at[rbase + j0 + 1], ixB, si)
        dA = issue(ixA, xi0, xj0, at0, s1, s2, s3)
        cpi.wait()
        dB = issue(ixB, xi1, xj1, at1, s4, s5, s6)
        dA[0].wait()
        dA[1].wait()
        dA[2].wait()
        scA = compute_scatter(ixA, xi0, xj0, at0, msg0)
        dB[0].wait()
        dB[1].wait()
        dB[2].wait()
        scB = compute_scatter(ixB, xi1, xj1, at1, msg1)
        scA.wait()
        cpi2 = pltpu.async_copy(idx3.at[rbase + j0 + 2], ixA, si)
        scB.wait()
        cpi2.wait()
        return ()

    lax.fori_loop(0, NCHUNK // 2, body2, ())
    dA = issue(ixA, xi0, xj0, at0, s1, s2, s3)
    dA[0].wait()
    dA[1].wait()
    dA[2].wait()
    compute_scatter(ixA, xi0, xj0, at0, msg0).wait()

    plsc.subcore_barrier()
    pltpu.sync_copy(U_sh.at[pl.ds(r0, ROWS_PER_S)],
                    out.at[pl.ds(c * NP + r0, ROWS_PER_S)])


def _edge_pass(xtab, atab, idx3, zeros_nuw):
    mesh = plsc.VectorSubcoreMesh(core_axis_name="c", subcore_axis_name="s")
    return pl.kernel(
        _edge_body,
        out_type=jax.ShapeDtypeStruct((2 * NP, UW), jnp.float32),
        mesh=mesh,
        scratch_types=[
            pltpu.VMEM((3, CHUNK), jnp.int32),
            pltpu.VMEM((3, CHUNK), jnp.int32),
            pltpu.VMEM((CHUNK, D), jnp.float32),
            pltpu.VMEM((CHUNK, D), jnp.float32),
            pltpu.VMEM((CHUNK, D), jnp.float32),
            pltpu.VMEM((CHUNK, D), jnp.float32),
            pltpu.VMEM((CHUNK, D), jnp.float32),
            pltpu.VMEM((CHUNK, D), jnp.float32),
            pltpu.VMEM((CHUNK, UW), jnp.float32),
            pltpu.VMEM((CHUNK, UW), jnp.float32),
            pltpu.VMEM((LANES, LANES), jnp.float32),
            pltpu.VMEM((LANES,), jnp.float32),
            pltpu.VMEM_SHARED((NP, UW), jnp.float32),
            pltpu.VMEM_SHARED((NREL, D), jnp.float32),
            pltpu.SemaphoreType.DMA,
            pltpu.SemaphoreType.DMA,
            pltpu.SemaphoreType.DMA,
            pltpu.SemaphoreType.DMA,
            pltpu.SemaphoreType.DMA,
            pltpu.SemaphoreType.DMA,
            pltpu.SemaphoreType.DMA,
            pltpu.SemaphoreType.DMA,
        ],
        compiler_params=pltpu.CompilerParams(
            needs_layout_passes=False, use_tc_tiling_on_sc=False),
    )(xtab, atab, idx3, zeros_nuw)


_BLK1 = 2000


def _mm1_body(x0_ref, w_ref, aself_ref, x1_ref, ui_ref):
    x0 = x0_ref[...]
    w = w_ref[0]
    y = jnp.dot(x0, w, preferred_element_type=jnp.float32)
    a = aself_ref[0]
    alpha = jnp.sum(y * y * a, axis=-1)
    alpha = jnp.where(alpha > 0, alpha, 0.2 * alpha)
    ee = jnp.exp(alpha)[:, None]
    x1_ref[0] = y
    ui_ref[0, :, :D] = ee * y
    lane16 = (lax.broadcasted_iota(jnp.int32, (1, 16), 1) == 0)
    ui_ref[0, :, D:UW] = ee * lane16.astype(jnp.float32)


def _mm1(x0, W1r, a1self):
    nb = N // _BLK1
    return pl.pallas_call(
        _mm1_body,
        grid=(H1, nb),
        in_specs=[
            pl.BlockSpec((_BLK1, D), lambda h, i: (i, 0)),
            pl.BlockSpec((1, D, D), lambda h, i: (h, 0, 0)),
            pl.BlockSpec((1, 1, D), lambda h, i: (h, 0, 0)),
        ],
        out_specs=[
            pl.BlockSpec((1, _BLK1, D), lambda h, i: (h, i, 0)),
            pl.BlockSpec((1, _BLK1, UW), lambda h, i: (h, i, 0)),
        ],
        out_shape=[
            jax.ShapeDtypeStruct((H1, N, D), jnp.float32),
            jax.ShapeDtypeStruct((H1, N, UW), jnp.float32),
        ],
    )(x0, W1r, a1self)


_BLK2 = 400


def _mid_body(ua_ref, ub_ref, ui_ref, b1_ref, w2_ref, a2_ref,
              x2_ref, ui2_ref):
    x2 = jnp.zeros((_BLK2, D), dtype=jnp.float32)
    for h in range(H1):
        S = ua_ref[h] + ub_ref[h] + ui_ref[h]
        num = S[:, :D]
        den = S[:, D:D + 1]
        o = num / (den + 1e-16) + b1_ref[h][None, :]
        hh = jnp.where(o > 0, o, 0.01 * o)
        x2 = x2 + jnp.dot(hh, w2_ref[h], preferred_element_type=jnp.float32)
    a2 = a2_ref[...]
    alpha = jnp.sum(x2 * x2 * a2, axis=-1)
    alpha = jnp.where(alpha > 0, alpha, 0.2 * alpha)
    ee = jnp.exp(alpha)[:, None]
    x2_ref[...] = x2
    ui2_ref[:, :D] = ee * x2
    lane16 = (lax.broadcasted_iota(jnp.int32, (1, 16), 1) == 0)
    ui2_ref[:, D:UW] = ee * lane16.astype(jnp.float32)


def _mid(Ua, Ub, uinit1, b1r, W2r, a2self):
    nb = N // _BLK2
    return pl.pallas_call(
        _mid_body,
        grid=(nb,),
        in_specs=[
            pl.BlockSpec((H1, _BLK2, UW), lambda i: (0, i, 0)),
            pl.BlockSpec((H1, _BLK2, UW), lambda i: (0, i, 0)),
            pl.BlockSpec((H1, _BLK2, UW), lambda i: (0, i, 0)),
            pl.BlockSpec((H1, D), lambda i: (0, 0)),
            pl.BlockSpec((H1, D, D), lambda i: (0, 0, 0)),
            pl.BlockSpec((1, D), lambda i: (0, 0)),
        ],
        out_specs=[
            pl.BlockSpec((_BLK2, D), lambda i: (i, 0)),
            pl.BlockSpec((_BLK2, UW), lambda i: (i, 0)),
        ],
        out_shape=[
            jax.ShapeDtypeStruct((N, D), jnp.float32),
            jax.ShapeDtypeStruct((N, UW), jnp.float32),
        ],
    )(Ua, Ub, uinit1, b1r, W2r, a2self)


_BLK3 = 2000


def _fin_body(ua_ref, ub_ref, ui_ref, b2_ref, out_ref):
    S = ua_ref[...] + ub_ref[...] + ui_ref[...]
    num = S[:, :D]
    den = S[:, D:D + 1]
    out_ref[...] = num / (den + 1e-16) + b2_ref[...]


def _fin(Ua, Ub, uinit2, b2):
    nb = N // _BLK3
    return pl.pallas_call(
        _fin_body,
        grid=(nb,),
        in_specs=[
            pl.BlockSpec((_BLK3, UW), lambda i: (i, 0)),
            pl.BlockSpec((_BLK3, UW), lambda i: (i, 0)),
            pl.BlockSpec((_BLK3, UW), lambda i: (i, 0)),
            pl.BlockSpec((1, D), lambda i: (0, 0)),
        ],
        out_specs=pl.BlockSpec((_BLK3, D), lambda i: (i, 0)),
        out_shape=jax.ShapeDtypeStruct((N, D), jnp.float32),
    )(Ua, Ub, uinit2, b2)


def kernel(entity, edge_index, edge_attr, emb_table, W1, att1, b1, W2, att2,
           b2):
    npad = EPT_PAD - EPT
    src = jnp.pad(edge_index[0].reshape(NC * NS, EPT), ((0, 0), (0, npad)),
                  constant_values=0).reshape(NC * NS, NCHUNK, CHUNK)
    dst = jnp.pad(edge_index[1].reshape(NC * NS, EPT), ((0, 0), (0, npad)),
                  constant_values=N).reshape(NC * NS, NCHUNK, CHUNK)
    ea = jnp.pad(edge_attr.reshape(NC * NS, EPT), ((0, 0), (0, npad)),
                 constant_values=0).reshape(NC * NS, NCHUNK, CHUNK)
    idx3 = jnp.stack([src, dst, ea], axis=2).reshape(
        NC * NS * NCHUNK, 3, CHUNK)
    zeros_nuw = jnp.zeros((NP, UW), dtype=jnp.float32)

    idx_pad = jnp.pad(entity, (0, _EMB_B - N))
    x0 = _emb_gather(emb_table, idx_pad)[:N]

    W1r = W1.reshape(D, H1, D).transpose(1, 0, 2)
    a1self = att1[NREL - 1].reshape(H1, 1, D)
    x1h, uinit1 = _mm1(x0, W1r, a1self)

    att1h = att1.transpose(1, 0, 2)
    U1 = [_edge_pass(x1h[h], att1h[h], idx3, zeros_nuw)
          for h in range(H1)]
    Ua = jnp.stack([u[:N] for u in U1])
    Ub = jnp.stack([u[NP:NP + N] for u in U1])

    b1r = b1.reshape(H1, D)
    W2r = W2.reshape(H1, D, D)
    a2self = att2[NREL - 1]
    x2, uinit2 = _mid(Ua, Ub, uinit1, b1r, W2r, a2self)

    U2 = _edge_pass(x2, att2[:, 0, :], idx3, zeros_nuw)
    out = _fin(U2[:N], U2[NP:NP + N], uinit2, b2.reshape(1, D))
    return out
